# baseline (speedup 1.0000x reference)
import jax
import jax.numpy as jnp
from jax.experimental import pallas as pl
from jax.experimental.pallas import tpu as pltpu

_B, _S, _H = 4, 4096, 1024
_HID = 512
_L = 119
_E = 128
_BLK = 512
_NREP = _S // _BLK
_NK = 4
_KC = _H // _NK


def _mlp_embed_broadcast(hs_ref, tab_ref, w1_ref, b1_ref, w2_ref, b2_ref,
                         out_ref, acc_ref, tile_ref, sem):
    k = pl.program_id(0)
    x = hs_ref[:, 7, :]
    part = jnp.dot(x, w1_ref[...], preferred_element_type=jnp.float32,
                   precision=jax.lax.Precision.HIGHEST)

    @pl.when(k == 0)
    def _init():
        acc_ref[...] = part

    @pl.when(k > 0)
    def _accum():
        acc_ref[...] += part

    @pl.when(k == _NK - 1)
    def _tail():
        h = acc_ref[...] + b1_ref[...]
        h = h * 0.5 * (1.0 + jax.lax.erf(h * 0.7071067811865476))
        logits = jnp.dot(h, w2_ref[...], preferred_element_type=jnp.float32,
                         precision=jax.lax.Precision.HIGHEST)
        logits = logits + b2_ref[...]
        m = jnp.max(logits, axis=-1, keepdims=True)
        iota = jax.lax.broadcasted_iota(jnp.int32, logits.shape, 1)
        cand = jnp.where(logits == m, iota, _L)
        idx = jnp.min(cand, axis=-1, keepdims=True)
        onehot = (iota == idx).astype(jnp.float32)
        emb = jnp.dot(onehot, tab_ref[...],
                      preferred_element_type=jnp.float32,
                      precision=jax.lax.Precision.HIGHEST)

        tile_ref[...] = jnp.broadcast_to(emb[:, None, :], (_B, _BLK, _E))
        copies = [
            pltpu.make_async_copy(
                tile_ref, out_ref.at[:, pl.ds(i * _BLK, _BLK), :], sem)
            for i in range(_NREP)
        ]
        for c in copies:
            c.start()
        for c in copies:
            c.wait()


def kernel(hidden_states, emb_table, W1, b1, W2, b2):
    out = pl.pallas_call(
        _mlp_embed_broadcast,
        grid=(_NK,),
        in_specs=[
            pl.BlockSpec((_B, 8, _KC), lambda k: (0, _S // 8 - 1, k)),
            pl.BlockSpec((_L, _E), lambda k: (0, 0)),
            pl.BlockSpec((_KC, _HID), lambda k: (k, 0)),
            pl.BlockSpec((1, _HID), lambda k: (0, 0)),
            pl.BlockSpec((_HID, _L), lambda k: (0, 0)),
            pl.BlockSpec((1, _L), lambda k: (0, 0)),
        ],
        out_specs=pl.BlockSpec(memory_space=pl.ANY),
        out_shape=jax.ShapeDtypeStruct((_B, _S, _E), jnp.float32),
        scratch_shapes=[
            pltpu.VMEM((_B, _HID), jnp.float32),
            pltpu.VMEM((_B, _BLK, _E), jnp.float32),
            pltpu.SemaphoreType.DMA,
        ],
    )(hidden_states, emb_table, W1, b1.reshape(1, _HID), W2,
      b2.reshape(1, _L))
    return out
